# baseline (speedup 1.0000x reference)
import functools

import jax
import jax.numpy as jnp
from jax import lax
from jax.experimental import pallas as pl
from jax.experimental.pallas import tpu as pltpu
from jax.experimental.pallas import tpu_sc as plsc

N_NODES = 10000
N_PAD = 10240
D = 128
E_HALF = 160000
E_TILE = 10000
C = 80
NCHUNK = E_TILE // C
STRIPE = N_PAD // 16


def _mm_kernel(x_ref, lr_ref, wl_ref, wi_ref, wo_ref, xwi_ref, xwo_ref, lp_ref):
    xb = x_ref[...]
    xwi_ref[...] = jnp.dot(xb, wi_ref[...], preferred_element_type=jnp.float32)
    xwo_ref[...] = jnp.dot(xb, wo_ref[...], preferred_element_type=jnp.float32)
    lp_ref[...] = jnp.dot(xb - lr_ref[...], wl_ref[...],
                          preferred_element_type=jnp.float32)


def _rel_kernel(re_ref, wi_ref, wo_ref, wr_ref, rwi_ref, rwo_ref, ro_ref):
    rb = re_ref[...]
    rwi_ref[...] = jnp.dot(rb, wi_ref[...], preferred_element_type=jnp.float32)
    rwo_ref[...] = jnp.dot(rb, wo_ref[...], preferred_element_type=jnp.float32)
    ro_ref[...] = jnp.dot(rb, wr_ref[...], preferred_element_type=jnp.float32)


def _bn_kernel(acc_ref, loop_ref, b_ref, g_ref, be_ref, out_ref):
    s = (acc_ref[0, :N_NODES, :] + acc_ref[1, :N_NODES, :]
         + loop_ref[...]) * (1.0 / 3.0) + b_ref[...][None, :]
    mean = jnp.mean(s, axis=0, keepdims=True)
    var = jnp.mean((s - mean) ** 2, axis=0, keepdims=True)
    out_ref[...] = ((s - mean) * lax.rsqrt(var + 1e-5)
                    * g_ref[...][None, :] + be_ref[...][None, :])


def _rsqrt16(x):
    s = (x + 1.0) * 0.5
    for _ in range(22):
        s = (s + x / s) * 0.5
    return jnp.where(x > 0.0, 1.0 / s, 0.0)


SCH = 400
NSUB = SCH // C
NSUP = E_TILE // SCH


def _sc_body(xw, rw, dst2d, srcf, etf, out,
             dst_sc, src_sc, et_sc, norm_c, deg_v,
             xbuf, rbuf, mbuf, ones_v,
             sem_x, sem_r, sem_s, sem_h, deg_sh, acc_sh):
    c = lax.axis_index("c")
    s = lax.axis_index("s")

    def fill1(i, _):
        ones_v[pl.ds(i * 16, 16)] = jnp.ones((16,), jnp.float32)
        norm_c[pl.ds(i * 16, 16)] = jnp.zeros((16,), jnp.float32)
        return 0
    lax.fori_loop(0, C // 16, fill1, 0)

    def fillx(i, _):
        for k in range(D // 16):
            xbuf[i, pl.ds(k * 16, 16)] = jnp.zeros((16,), jnp.float32)
        return 0
    lax.fori_loop(0, C, fillx, 0)

    for b in range(STRIPE // C):
        pltpu.sync_copy(norm_c, deg_sh.at[pl.ds(s * STRIPE + b * C, C)])
        pltpu.sync_copy(xbuf, acc_sh.at[pl.ds(s * STRIPE + b * C, C)])
    plsc.subcore_barrier()

    def hist_sup(g, _):
        pltpu.sync_copy(dst2d.at[c, s, g], dst_sc)

        def hist(j, _):
            pltpu.async_copy(ones_v.at[pl.ds(0, C)], deg_sh.at[dst_sc.at[j]],
                             sem_h, add=True)
            return 0
        lax.fori_loop(0, NSUB, hist, 0)

        def drain(j, _):
            pltpu.make_async_copy(ones_v.at[pl.ds(0, C)],
                                  deg_sh.at[dst_sc.at[j]], sem_h).wait()
            return 0
        lax.fori_loop(0, NSUB, drain, 0)
        return 0
    lax.fori_loop(0, NSUP, hist_sup, 0)
    plsc.subcore_barrier()

    pltpu.sync_copy(deg_sh.at[pl.ds(s * STRIPE, STRIPE)],
                    deg_v.at[pl.ds(0, STRIPE)])

    def inv(i, _):
        sl = pl.ds(i * 16, 16)
        deg_v[sl] = _rsqrt16(deg_v[sl])
        return 0
    lax.fori_loop(0, STRIPE // 16, inv, 0)
    pltpu.sync_copy(deg_v.at[pl.ds(0, STRIPE)],
                    deg_sh.at[pl.ds(s * STRIPE, STRIPE)])
    plsc.subcore_barrier()
    pltpu.sync_copy(deg_sh.at[pl.ds(0, N_NODES)], deg_v)

    def sup(g, _):
        pltpu.sync_copy(dst2d.at[c, s, g], dst_sc)
        pltpu.sync_copy(srcf.at[c, s, g], src_sc)
        pltpu.sync_copy(etf.at[c, s, g], et_sc)

        def chunk(j, _):
            dx = pltpu.async_copy(xw.at[src_sc.at[pl.ds(j * C, C)]], xbuf,
                                  sem_x)
            dr = pltpu.async_copy(rw.at[et_sc.at[pl.ds(j * C, C)]], rbuf,
                                  sem_r)

            for gg in range(C // 16):
                sl16 = pl.ds(gg * 16, 16)
                i0 = dst_sc[j, sl16]
                s16 = src_sc[pl.ds(j * C + gg * 16, 16)]
                d0 = plsc.load_gather(deg_v, [i0])
                d1 = plsc.load_gather(deg_v, [s16 - c * N_NODES])
                norm_c[sl16] = d0 * d1

            @pl.when(g * NSUB + j > 0)
            def _():
                pltpu.make_async_copy(mbuf, acc_sh.at[dst_sc.at[j]],
                                      sem_s).wait()
            dx.wait()
            dr.wait()

            def edge(e, _):
                n = plsc.load_gather(norm_c, [jnp.full((16,), e, jnp.int32)])
                for k in range(D // 16):
                    sl = pl.ds(k * 16, 16)
                    mbuf[e, sl] = (xbuf[e, sl] - rbuf[e, sl]) * n
                return 0
            lax.fori_loop(0, C, edge, 0)
            pltpu.async_copy(mbuf, acc_sh.at[dst_sc.at[j]], sem_s, add=True)
            return 0
        lax.fori_loop(0, NSUB, chunk, 0)
        return 0
    lax.fori_loop(0, NSUP, sup, 0)
    pltpu.make_async_copy(mbuf, acc_sh.at[dst_sc.at[NSUB - 1]], sem_s).wait()
    plsc.subcore_barrier()

    pltpu.sync_copy(acc_sh.at[pl.ds(s * STRIPE, STRIPE)],
                    out.at[c, pl.ds(s * STRIPE, STRIPE)])


def _sc_call(xw_cat, rw_cat, dst2d, srcf, etf):
    mesh = plsc.VectorSubcoreMesh(core_axis_name="c", subcore_axis_name="s",
                                  num_cores=2, num_subcores=16)
    return pl.kernel(
        _sc_body,
        out_type=jax.ShapeDtypeStruct((2, N_PAD, D), jnp.float32),
        mesh=mesh,
        compiler_params=pltpu.CompilerParams(needs_layout_passes=False),
        scratch_types=[
            pltpu.VMEM((NSUB, C), jnp.int32),
            pltpu.VMEM((SCH,), jnp.int32),
            pltpu.VMEM((SCH,), jnp.int32),
            pltpu.VMEM((C,), jnp.float32),
            pltpu.VMEM((N_NODES,), jnp.float32),
            pltpu.VMEM((C, D), jnp.float32),
            pltpu.VMEM((C, D), jnp.float32),
            pltpu.VMEM((C, D), jnp.float32),
            pltpu.VMEM((C,), jnp.float32),
            pltpu.SemaphoreType.DMA,
            pltpu.SemaphoreType.DMA,
            pltpu.SemaphoreType.DMA,
            pltpu.SemaphoreType.DMA,
            pltpu.VMEM_SHARED((N_PAD,), jnp.float32),
            pltpu.VMEM_SHARED((N_PAD, D), jnp.float32),
        ],
    )(xw_cat, rw_cat, dst2d, srcf, etf)


def kernel(x, edge_index, edge_type, rel_embed, w_loop, w_in, w_out, w_rel,
           loop_rel, bias, bn_gamma, bn_beta):
    nb = 10
    xw_in, xw_out, loop_res = pl.pallas_call(
        _mm_kernel,
        grid=(nb,),
        in_specs=[
            pl.BlockSpec((N_NODES // nb, D), lambda i: (i, 0)),
            pl.BlockSpec((1, D), lambda i: (0, 0)),
            pl.BlockSpec((D, D), lambda i: (0, 0)),
            pl.BlockSpec((D, D), lambda i: (0, 0)),
            pl.BlockSpec((D, D), lambda i: (0, 0)),
        ],
        out_specs=[
            pl.BlockSpec((N_NODES // nb, D), lambda i: (i, 0)),
            pl.BlockSpec((N_NODES // nb, D), lambda i: (i, 0)),
            pl.BlockSpec((N_NODES // nb, D), lambda i: (i, 0)),
        ],
        out_shape=[jax.ShapeDtypeStruct((N_NODES, D), jnp.float32)] * 3,
    )(x, loop_rel, w_loop, w_in, w_out)

    rw_in, rw_out, rel_out = pl.pallas_call(
        _rel_kernel,
        out_shape=[jax.ShapeDtypeStruct((rel_embed.shape[0], D), jnp.float32)] * 3,
    )(rel_embed, w_in, w_out, w_rel)

    half_off = jnp.array([0, N_NODES], jnp.int32)[:, None, None]
    rel_off = jnp.array([0, rel_embed.shape[0]], jnp.int32)[:, None, None]
    srcf = (edge_index[1].reshape(2, 16, E_TILE)
            + half_off).reshape(2, 16, NSUP, SCH)
    etf = (edge_type.reshape(2, 16, E_TILE)
           + rel_off).reshape(2, 16, NSUP, SCH)
    dst2d = edge_index[0].reshape(2, 16, NSUP, NSUB, C)
    xw_cat = jnp.concatenate([xw_in, xw_out], axis=0)
    rw_cat = jnp.concatenate([rw_in, rw_out], axis=0)

    acc = _sc_call(xw_cat, rw_cat, dst2d, srcf, etf)

    out = pl.pallas_call(
        _bn_kernel,
        out_shape=jax.ShapeDtypeStruct((N_NODES, D), jnp.float32),
    )(acc, loop_res, bias, bn_gamma, bn_beta)
    return out, rel_out


# speedup vs baseline: 1.0282x; 1.0282x over previous
import functools

import jax
import jax.numpy as jnp
from jax import lax
from jax.experimental import pallas as pl
from jax.experimental.pallas import tpu as pltpu
from jax.experimental.pallas import tpu_sc as plsc

N_NODES = 10000
N_PAD = 10240
D = 128
E_HALF = 160000
E_TILE = 10000
C = 80
NCHUNK = E_TILE // C
STRIPE = N_PAD // 16


def _mm_kernel(x_ref, lr_ref, wl_ref, wi_ref, wo_ref, xwi_ref, xwo_ref, lp_ref):
    xb = x_ref[...]
    xwi_ref[...] = jnp.dot(xb, wi_ref[...], preferred_element_type=jnp.float32)
    xwo_ref[...] = jnp.dot(xb, wo_ref[...], preferred_element_type=jnp.float32)
    lp_ref[...] = jnp.dot(xb - lr_ref[...], wl_ref[...],
                          preferred_element_type=jnp.float32)


def _rel_kernel(re_ref, wi_ref, wo_ref, wr_ref, rwi_ref, rwo_ref, ro_ref):
    rb = re_ref[...]
    rwi_ref[...] = jnp.dot(rb, wi_ref[...], preferred_element_type=jnp.float32)
    rwo_ref[...] = jnp.dot(rb, wo_ref[...], preferred_element_type=jnp.float32)
    ro_ref[...] = jnp.dot(rb, wr_ref[...], preferred_element_type=jnp.float32)


def _bn_kernel(acc_ref, loop_ref, b_ref, g_ref, be_ref, out_ref):
    s = (acc_ref[0, :N_NODES, :] + acc_ref[1, :N_NODES, :]
         + loop_ref[...]) * (1.0 / 3.0) + b_ref[...][None, :]
    mean = jnp.mean(s, axis=0, keepdims=True)
    var = jnp.mean((s - mean) ** 2, axis=0, keepdims=True)
    out_ref[...] = ((s - mean) * lax.rsqrt(var + 1e-5)
                    * g_ref[...][None, :] + be_ref[...][None, :])


def _rsqrt16(x):
    s = (x + 1.0) * 0.5
    for _ in range(22):
        s = (s + x / s) * 0.5
    return jnp.where(x > 0.0, 1.0 / s, 0.0)


SCH = 2000
NSUB = SCH // C
NSUP = E_TILE // SCH


def _sc_body(xw, rw, dst2d, srcf, etf, out,
             dst_sc, src_sc, et_sc, norm_c, deg_v,
             xbuf, rbuf, ones_v,
             sem_x, sem_r, deg_sh, acc_sh):
    c = lax.axis_index("c")
    s = lax.axis_index("s")

    def fill1(i, _):
        ones_v[pl.ds(i * 16, 16)] = jnp.ones((16,), jnp.float32)
        norm_c[pl.ds(i * 16, 16)] = jnp.zeros((16,), jnp.float32)
        return 0
    lax.fori_loop(0, C // 16, fill1, 0)

    def fillx(i, _):
        for k in range(D // 16):
            xbuf[i, pl.ds(k * 16, 16)] = jnp.zeros((16,), jnp.float32)
        return 0
    lax.fori_loop(0, C, fillx, 0)

    for b in range(STRIPE // C):
        pltpu.sync_copy(norm_c, deg_sh.at[pl.ds(s * STRIPE + b * C, C)])
        pltpu.sync_copy(xbuf, acc_sh.at[pl.ds(s * STRIPE + b * C, C)])
    plsc.subcore_barrier()

    def hist_sup(g, _):
        pltpu.sync_copy(dst2d.at[c, s, g], dst_sc)

        def hist(j, _):
            pltpu.sync_copy(ones_v.at[pl.ds(0, C)], deg_sh.at[dst_sc.at[j]],
                            add=True)
            return 0
        lax.fori_loop(0, NSUB, hist, 0)
        return 0
    lax.fori_loop(0, NSUP, hist_sup, 0)
    plsc.subcore_barrier()

    pltpu.sync_copy(deg_sh.at[pl.ds(s * STRIPE, STRIPE)],
                    deg_v.at[pl.ds(0, STRIPE)])

    def inv(i, _):
        sl = pl.ds(i * 16, 16)
        deg_v[sl] = _rsqrt16(deg_v[sl])
        return 0
    lax.fori_loop(0, STRIPE // 16, inv, 0)
    pltpu.sync_copy(deg_v.at[pl.ds(0, STRIPE)],
                    deg_sh.at[pl.ds(s * STRIPE, STRIPE)])
    plsc.subcore_barrier()
    pltpu.sync_copy(deg_sh.at[pl.ds(0, N_NODES)], deg_v)

    def sup(g, _):
        pltpu.sync_copy(dst2d.at[c, s, g], dst_sc)
        pltpu.sync_copy(srcf.at[c, s, g], src_sc)
        pltpu.sync_copy(etf.at[c, s, g], et_sc)

        def chunk(j, _):
            dx = pltpu.async_copy(xw.at[src_sc.at[pl.ds(j * C, C)]], xbuf,
                                  sem_x)
            dr = pltpu.async_copy(rw.at[et_sc.at[pl.ds(j * C, C)]], rbuf,
                                  sem_r)

            for gg in range(C // 16):
                sl16 = pl.ds(gg * 16, 16)
                i0 = dst_sc[j, sl16]
                s16 = src_sc[pl.ds(j * C + gg * 16, 16)]
                d0 = plsc.load_gather(deg_v, [i0])
                d1 = plsc.load_gather(deg_v, [s16 - c * N_NODES])
                norm_c[sl16] = d0 * d1

            dx.wait()
            dr.wait()

            def edge(e, _):
                n = plsc.load_gather(norm_c, [jnp.full((16,), e, jnp.int32)])
                for k in range(D // 16):
                    sl = pl.ds(k * 16, 16)
                    xbuf[e, sl] = (xbuf[e, sl] - rbuf[e, sl]) * n
                return 0
            lax.fori_loop(0, C, edge, 0)
            pltpu.sync_copy(xbuf, acc_sh.at[dst_sc.at[j]], add=True)
            return 0
        lax.fori_loop(0, NSUB, chunk, 0)
        return 0
    lax.fori_loop(0, NSUP, sup, 0)
    plsc.subcore_barrier()

    pltpu.sync_copy(acc_sh.at[pl.ds(s * STRIPE, STRIPE)],
                    out.at[c, pl.ds(s * STRIPE, STRIPE)])


def _sc_call(xw_cat, rw_cat, dst2d, srcf, etf):
    mesh = plsc.VectorSubcoreMesh(core_axis_name="c", subcore_axis_name="s",
                                  num_cores=2, num_subcores=16)
    return pl.kernel(
        _sc_body,
        out_type=jax.ShapeDtypeStruct((2, N_PAD, D), jnp.float32),
        mesh=mesh,
        compiler_params=pltpu.CompilerParams(needs_layout_passes=False),
        scratch_types=[
            pltpu.VMEM((NSUB, C), jnp.int32),
            pltpu.VMEM((SCH,), jnp.int32),
            pltpu.VMEM((SCH,), jnp.int32),
            pltpu.VMEM((C,), jnp.float32),
            pltpu.VMEM((N_NODES,), jnp.float32),
            pltpu.VMEM((C, D), jnp.float32),
            pltpu.VMEM((C, D), jnp.float32),
            pltpu.VMEM((C,), jnp.float32),
            pltpu.SemaphoreType.DMA,
            pltpu.SemaphoreType.DMA,
            pltpu.VMEM_SHARED((N_PAD,), jnp.float32),
            pltpu.VMEM_SHARED((N_PAD, D), jnp.float32),
        ],
    )(xw_cat, rw_cat, dst2d, srcf, etf)


def kernel(x, edge_index, edge_type, rel_embed, w_loop, w_in, w_out, w_rel,
           loop_rel, bias, bn_gamma, bn_beta):
    nb = 10
    xw_in, xw_out, loop_res = pl.pallas_call(
        _mm_kernel,
        grid=(nb,),
        in_specs=[
            pl.BlockSpec((N_NODES // nb, D), lambda i: (i, 0)),
            pl.BlockSpec((1, D), lambda i: (0, 0)),
            pl.BlockSpec((D, D), lambda i: (0, 0)),
            pl.BlockSpec((D, D), lambda i: (0, 0)),
            pl.BlockSpec((D, D), lambda i: (0, 0)),
        ],
        out_specs=[
            pl.BlockSpec((N_NODES // nb, D), lambda i: (i, 0)),
            pl.BlockSpec((N_NODES // nb, D), lambda i: (i, 0)),
            pl.BlockSpec((N_NODES // nb, D), lambda i: (i, 0)),
        ],
        out_shape=[jax.ShapeDtypeStruct((N_NODES, D), jnp.float32)] * 3,
    )(x, loop_rel, w_loop, w_in, w_out)

    rw_in, rw_out, rel_out = pl.pallas_call(
        _rel_kernel,
        out_shape=[jax.ShapeDtypeStruct((rel_embed.shape[0], D), jnp.float32)] * 3,
    )(rel_embed, w_in, w_out, w_rel)

    half_off = jnp.array([0, N_NODES], jnp.int32)[:, None, None]
    rel_off = jnp.array([0, rel_embed.shape[0]], jnp.int32)[:, None, None]
    srcf = (edge_index[1].reshape(2, 16, E_TILE)
            + half_off).reshape(2, 16, NSUP, SCH)
    etf = (edge_type.reshape(2, 16, E_TILE)
           + rel_off).reshape(2, 16, NSUP, SCH)
    dst2d = edge_index[0].reshape(2, 16, NSUP, NSUB, C)
    xw_cat = jnp.concatenate([xw_in, xw_out], axis=0)
    rw_cat = jnp.concatenate([rw_in, rw_out], axis=0)

    acc = _sc_call(xw_cat, rw_cat, dst2d, srcf, etf)

    out = pl.pallas_call(
        _bn_kernel,
        out_shape=jax.ShapeDtypeStruct((N_NODES, D), jnp.float32),
    )(acc, loop_res, bias, bn_gamma, bn_beta)
    return out, rel_out


# speedup vs baseline: 1.4851x; 1.4444x over previous
import functools

import jax
import jax.numpy as jnp
from jax import lax
from jax.experimental import pallas as pl
from jax.experimental.pallas import tpu as pltpu
from jax.experimental.pallas import tpu_sc as plsc

N_NODES = 10000
N_PAD = 10240
D = 128
E_HALF = 160000
E_TILE = 10000
E_TILE_P = 10080
C = 112
STRIPE = N_PAD // 16


def _mm_kernel(x_ref, lr_ref, wl_ref, wi_ref, wo_ref, xwi_ref, xwo_ref, lp_ref):
    xb = x_ref[...]
    xwi_ref[...] = jnp.dot(xb, wi_ref[...], preferred_element_type=jnp.float32)
    xwo_ref[...] = jnp.dot(xb, wo_ref[...], preferred_element_type=jnp.float32)
    lp_ref[...] = jnp.dot(xb - lr_ref[...], wl_ref[...],
                          preferred_element_type=jnp.float32)


def _rel_kernel(re_ref, wi_ref, wo_ref, wr_ref, rwi_ref, rwo_ref, ro_ref):
    rb = re_ref[...]
    rwi_ref[...] = jnp.dot(rb, wi_ref[...], preferred_element_type=jnp.float32)
    rwo_ref[...] = jnp.dot(rb, wo_ref[...], preferred_element_type=jnp.float32)
    ro_ref[...] = jnp.dot(rb, wr_ref[...], preferred_element_type=jnp.float32)


def _bn_kernel(acc_ref, loop_ref, b_ref, g_ref, be_ref, out_ref):
    s = (acc_ref[0, :N_NODES, :] + acc_ref[1, :N_NODES, :]
         + loop_ref[...]) * (1.0 / 3.0) + b_ref[...][None, :]
    mean = jnp.mean(s, axis=0, keepdims=True)
    var = jnp.mean((s - mean) ** 2, axis=0, keepdims=True)
    out_ref[...] = ((s - mean) * lax.rsqrt(var + 1e-5)
                    * g_ref[...][None, :] + be_ref[...][None, :])


def _rsqrt16(x):
    s = (x + 1.0) * 0.5
    for _ in range(22):
        s = (s + x / s) * 0.5
    return jnp.where(x > 0.0, 1.0 / s, 0.0)


SCH = 1120
NSUB = SCH // C
NSUP = E_TILE_P // SCH


def _sc_body(xw, rw, dst2d, srcf, etf, out,
             dst_sc, src_sc, et_sc, norm_sc, deg_v,
             xbuf, rbuf, ones_v, zs_v, deg_sh, acc_sh):
    c = lax.axis_index("c")
    s = lax.axis_index("s")

    def fill(i, _):
        zs_v[pl.ds(i * 16, 16)] = jnp.zeros((16,), jnp.float32)
        return 0
    lax.fori_loop(0, STRIPE // 16, fill, 0)

    def fill1(i, _):
        ones_v[pl.ds(i * 16, 16)] = jnp.ones((16,), jnp.float32)
        return 0
    lax.fori_loop(0, 7, fill1, 0)

    def fillx(i, _):
        for k in range(D // 16):
            xbuf[i, pl.ds(k * 16, 16)] = jnp.zeros((16,), jnp.float32)
        return 0
    lax.fori_loop(0, C, fillx, 0)

    pltpu.sync_copy(zs_v, deg_sh.at[pl.ds(s * STRIPE, STRIPE)])
    for b in range(STRIPE // C):
        pltpu.sync_copy(xbuf, acc_sh.at[pl.ds(s * STRIPE + b * C, C)])
    rem = STRIPE - (STRIPE // C) * C
    pltpu.sync_copy(xbuf.at[pl.ds(0, rem)],
                    acc_sh.at[pl.ds(s * STRIPE + (STRIPE // C) * C, rem)])
    plsc.subcore_barrier()

    def hist_sup(g, _):
        pltpu.sync_copy(dst2d.at[c, s, g], dst_sc)

        def hist(j, _):
            pltpu.sync_copy(ones_v.at[pl.ds(0, C)], deg_sh.at[dst_sc.at[j]],
                            add=True)
            return 0
        lax.fori_loop(0, NSUB, hist, 0)
        return 0
    lax.fori_loop(0, NSUP, hist_sup, 0)
    plsc.subcore_barrier()

    pltpu.sync_copy(deg_sh.at[pl.ds(s * STRIPE, STRIPE)],
                    deg_v.at[pl.ds(0, STRIPE)])

    def inv(i, _):
        sl = pl.ds(i * 16, 16)
        deg_v[sl] = _rsqrt16(deg_v[sl])
        return 0
    lax.fori_loop(0, STRIPE // 16, inv, 0)
    pltpu.sync_copy(deg_v.at[pl.ds(0, STRIPE)],
                    deg_sh.at[pl.ds(s * STRIPE, STRIPE)])
    plsc.subcore_barrier()
    pltpu.sync_copy(deg_sh, deg_v)

    def sup(g, _):
        pltpu.sync_copy(dst2d.at[c, s, g], dst_sc)
        pltpu.sync_copy(srcf.at[c, s, g], src_sc)
        pltpu.sync_copy(etf.at[c, s, g], et_sc)

        def nrm(j, _):
            for gg in range(C // 16):
                i0 = dst_sc[j, pl.ds(gg * 16, 16)]
                sl = pl.ds(j * C + gg * 16, 16)
                d0 = plsc.load_gather(deg_v, [i0])
                d1 = plsc.load_gather(deg_v, [src_sc[sl] - c * N_NODES])
                norm_sc[sl] = d0 * d1
            return 0
        lax.fori_loop(0, NSUB, nrm, 0)

        def chunk(j, _):
            pltpu.sync_copy(xw.at[src_sc.at[pl.ds(j * C, C)]], xbuf)
            pltpu.sync_copy(rw.at[et_sc.at[pl.ds(j * C, C)]], rbuf)

            def edge(e, _):
                n = plsc.load_gather(norm_sc, [jnp.full((16,), j * C + e,
                                                        jnp.int32)])
                for k in range(D // 16):
                    sl = pl.ds(k * 16, 16)
                    xbuf[e, sl] = (xbuf[e, sl] - rbuf[e, sl]) * n
                return 0
            lax.fori_loop(0, C, edge, 0)
            pltpu.sync_copy(xbuf, acc_sh.at[dst_sc.at[j]], add=True)
            return 0
        lax.fori_loop(0, NSUB, chunk, 0)
        return 0
    lax.fori_loop(0, NSUP, sup, 0)
    plsc.subcore_barrier()

    pltpu.sync_copy(acc_sh.at[pl.ds(s * STRIPE, STRIPE)],
                    out.at[c, pl.ds(s * STRIPE, STRIPE)])


def _sc_call(xw_cat, rw_cat, dst2d, srcf, etf):
    mesh = plsc.VectorSubcoreMesh(core_axis_name="c", subcore_axis_name="s",
                                  num_cores=2, num_subcores=16)
    return pl.kernel(
        _sc_body,
        out_type=jax.ShapeDtypeStruct((2, N_PAD, D), jnp.float32),
        mesh=mesh,
        compiler_params=pltpu.CompilerParams(needs_layout_passes=False),
        scratch_types=[
            pltpu.VMEM((NSUB, C), jnp.int32),
            pltpu.VMEM((SCH,), jnp.int32),
            pltpu.VMEM((SCH,), jnp.int32),
            pltpu.VMEM((SCH,), jnp.float32),
            pltpu.VMEM((N_PAD,), jnp.float32),
            pltpu.VMEM((C, D), jnp.float32),
            pltpu.VMEM((C, D), jnp.float32),
            pltpu.VMEM((112,), jnp.float32),
            pltpu.VMEM((STRIPE,), jnp.float32),
            pltpu.VMEM_SHARED((N_PAD,), jnp.float32),
            pltpu.VMEM_SHARED((N_PAD, D), jnp.float32),
        ],
    )(xw_cat, rw_cat, dst2d, srcf, etf)


def kernel(x, edge_index, edge_type, rel_embed, w_loop, w_in, w_out, w_rel,
           loop_rel, bias, bn_gamma, bn_beta):
    nb = 10
    xw_in, xw_out, loop_res = pl.pallas_call(
        _mm_kernel,
        grid=(nb,),
        in_specs=[
            pl.BlockSpec((N_NODES // nb, D), lambda i: (i, 0)),
            pl.BlockSpec((1, D), lambda i: (0, 0)),
            pl.BlockSpec((D, D), lambda i: (0, 0)),
            pl.BlockSpec((D, D), lambda i: (0, 0)),
            pl.BlockSpec((D, D), lambda i: (0, 0)),
        ],
        out_specs=[
            pl.BlockSpec((N_NODES // nb, D), lambda i: (i, 0)),
            pl.BlockSpec((N_NODES // nb, D), lambda i: (i, 0)),
            pl.BlockSpec((N_NODES // nb, D), lambda i: (i, 0)),
        ],
        out_shape=[jax.ShapeDtypeStruct((N_NODES, D), jnp.float32)] * 3,
    )(x, loop_rel, w_loop, w_in, w_out)

    rw_in, rw_out, rel_out = pl.pallas_call(
        _rel_kernel,
        out_shape=[jax.ShapeDtypeStruct((rel_embed.shape[0], D), jnp.float32)] * 3,
    )(rel_embed, w_in, w_out, w_rel)

    half_off = jnp.array([0, N_NODES], jnp.int32)[:, None, None]
    rel_off = jnp.array([0, rel_embed.shape[0]], jnp.int32)[:, None, None]
    npad = E_TILE_P - E_TILE
    src_p = jnp.full((2, 16, npad), 0, jnp.int32) + half_off
    et_p = jnp.full((2, 16, npad), 0, jnp.int32) + rel_off
    dst_p = jnp.full((2, 16, npad), N_NODES, jnp.int32)
    srcf = jnp.concatenate(
        [edge_index[1].reshape(2, 16, E_TILE) + half_off, src_p],
        axis=2).reshape(2, 16, NSUP, SCH)
    etf = jnp.concatenate(
        [edge_type.reshape(2, 16, E_TILE) + rel_off, et_p],
        axis=2).reshape(2, 16, NSUP, SCH)
    dst2d = jnp.concatenate(
        [edge_index[0].reshape(2, 16, E_TILE), dst_p],
        axis=2).reshape(2, 16, NSUP, NSUB, C)
    xw_cat = jnp.concatenate([xw_in, xw_out], axis=0)
    rw_cat = jnp.concatenate([rw_in, rw_out], axis=0)

    acc = _sc_call(xw_cat, rw_cat, dst2d, srcf, etf)

    out = pl.pallas_call(
        _bn_kernel,
        out_shape=jax.ShapeDtypeStruct((N_NODES, D), jnp.float32),
    )(acc, loop_res, bias, bn_gamma, bn_beta)
    return out, rel_out


# speedup vs baseline: 1.6905x; 1.1383x over previous
import functools

import jax
import jax.numpy as jnp
from jax import lax
from jax.experimental import pallas as pl
from jax.experimental.pallas import tpu as pltpu
from jax.experimental.pallas import tpu_sc as plsc

N_NODES = 10000
N_PAD = 10240
D = 128
E_HALF = 160000
E_TILE = 10000
C = 80
NCHUNK = E_TILE // C
STRIPE = N_PAD // 16


def _mm_kernel(x_ref, lr_ref, wl_ref, wi_ref, wo_ref, xwi_ref, xwo_ref, lp_ref):
    xb = x_ref[...]
    xwi_ref[...] = jnp.dot(xb, wi_ref[...], preferred_element_type=jnp.float32)
    xwo_ref[...] = jnp.dot(xb, wo_ref[...], preferred_element_type=jnp.float32)
    lp_ref[...] = jnp.dot(xb - lr_ref[...], wl_ref[...],
                          preferred_element_type=jnp.float32)


def _rel_kernel(re_ref, wi_ref, wo_ref, wr_ref, rwi_ref, rwo_ref, ro_ref):
    rb = re_ref[...]
    rwi_ref[...] = jnp.dot(rb, wi_ref[...], preferred_element_type=jnp.float32)
    rwo_ref[...] = jnp.dot(rb, wo_ref[...], preferred_element_type=jnp.float32)
    ro_ref[...] = jnp.dot(rb, wr_ref[...], preferred_element_type=jnp.float32)


def _bn_kernel(acc_ref, loop_ref, b_ref, g_ref, be_ref, out_ref):
    s = (acc_ref[0, :N_NODES, :] + acc_ref[1, :N_NODES, :]
         + loop_ref[...]) * (1.0 / 3.0) + b_ref[...][None, :]
    mean = jnp.mean(s, axis=0, keepdims=True)
    var = jnp.mean((s - mean) ** 2, axis=0, keepdims=True)
    out_ref[...] = ((s - mean) * lax.rsqrt(var + 1e-5)
                    * g_ref[...][None, :] + be_ref[...][None, :])


def _rsqrt16(x):
    s = (x + 1.0) * 0.5
    for _ in range(22):
        s = (s + x / s) * 0.5
    return jnp.where(x > 0.0, 1.0 / s, 0.0)


SCH = 2000
NSUB = SCH // C
NSUP = E_TILE // SCH


R_PAD = 512


def _sc_body(xw, rw, dst2d, srcf, etf, out,
             dst_sc, src_sc, et_sc, norm_sc, deg_v,
             xbuf, rbuf, ones_v, zs_v, rw_sh, deg_sh, acc_sh):
    c = lax.axis_index("c")
    s = lax.axis_index("s")

    pltpu.sync_copy(rw.at[c, pl.ds(s * (R_PAD // 16), R_PAD // 16)],
                    rw_sh.at[pl.ds(s * (R_PAD // 16), R_PAD // 16)])

    def fill(i, _):
        zs_v[pl.ds(i * 16, 16)] = jnp.zeros((16,), jnp.float32)
        return 0
    lax.fori_loop(0, STRIPE // 16, fill, 0)

    def fill1(i, _):
        ones_v[pl.ds(i * 16, 16)] = jnp.ones((16,), jnp.float32)
        return 0
    lax.fori_loop(0, 7, fill1, 0)

    def fillx(i, _):
        for k in range(D // 16):
            xbuf[i, pl.ds(k * 16, 16)] = jnp.zeros((16,), jnp.float32)
        return 0
    lax.fori_loop(0, C, fillx, 0)

    pltpu.sync_copy(zs_v, deg_sh.at[pl.ds(s * STRIPE, STRIPE)])
    for b in range(STRIPE // C):
        pltpu.sync_copy(xbuf, acc_sh.at[pl.ds(s * STRIPE + b * C, C)])
    plsc.subcore_barrier()

    def hist_sup(g, _):
        pltpu.sync_copy(dst2d.at[c, s, g], dst_sc)

        def hist(j, _):
            pltpu.sync_copy(ones_v.at[pl.ds(0, C)], deg_sh.at[dst_sc.at[j]],
                            add=True)
            return 0
        lax.fori_loop(0, NSUB, hist, 0)
        return 0
    lax.fori_loop(0, NSUP, hist_sup, 0)
    plsc.subcore_barrier()

    pltpu.sync_copy(deg_sh.at[pl.ds(s * STRIPE, STRIPE)],
                    deg_v.at[pl.ds(0, STRIPE)])

    def inv(i, _):
        sl = pl.ds(i * 16, 16)
        deg_v[sl] = _rsqrt16(deg_v[sl])
        return 0
    lax.fori_loop(0, STRIPE // 16, inv, 0)
    pltpu.sync_copy(deg_v.at[pl.ds(0, STRIPE)],
                    deg_sh.at[pl.ds(s * STRIPE, STRIPE)])
    plsc.subcore_barrier()
    pltpu.sync_copy(deg_sh, deg_v)

    def sup(g, _):
        pltpu.sync_copy(dst2d.at[c, s, g], dst_sc)
        pltpu.sync_copy(srcf.at[c, s, g], src_sc)
        pltpu.sync_copy(etf.at[c, s, g], et_sc)

        def nrm(j, _):
            for gg in range(C // 16):
                i0 = dst_sc[j, pl.ds(gg * 16, 16)]
                sl = pl.ds(j * C + gg * 16, 16)
                d0 = plsc.load_gather(deg_v, [i0])
                d1 = plsc.load_gather(deg_v, [src_sc[sl] - c * N_NODES])
                norm_sc[sl] = d0 * d1
            return 0
        lax.fori_loop(0, NSUB, nrm, 0)

        def chunk(j, _):
            pltpu.sync_copy(xw.at[src_sc.at[pl.ds(j * C, C)]], xbuf)
            pltpu.sync_copy(rw_sh.at[et_sc.at[pl.ds(j * C, C)]], rbuf)

            def edge(e, _):
                n = plsc.load_gather(norm_sc, [jnp.full((16,), j * C + e,
                                                        jnp.int32)])
                for k in range(D // 16):
                    sl = pl.ds(k * 16, 16)
                    xbuf[e, sl] = (xbuf[e, sl] - rbuf[e, sl]) * n
                return 0
            lax.fori_loop(0, C, edge, 0)
            pltpu.sync_copy(xbuf, acc_sh.at[dst_sc.at[j]], add=True)
            return 0
        lax.fori_loop(0, NSUB, chunk, 0)
        return 0
    lax.fori_loop(0, NSUP, sup, 0)
    plsc.subcore_barrier()

    pltpu.sync_copy(acc_sh.at[pl.ds(s * STRIPE, STRIPE)],
                    out.at[c, pl.ds(s * STRIPE, STRIPE)])


def _sc_call(xw_cat, rw_cat, dst2d, srcf, etf):
    mesh = plsc.VectorSubcoreMesh(core_axis_name="c", subcore_axis_name="s",
                                  num_cores=2, num_subcores=16)
    return pl.kernel(
        _sc_body,
        out_type=jax.ShapeDtypeStruct((2, N_PAD, D), jnp.float32),
        mesh=mesh,
        compiler_params=pltpu.CompilerParams(needs_layout_passes=False),
        scratch_types=[
            pltpu.VMEM((NSUB, C), jnp.int32),
            pltpu.VMEM((SCH,), jnp.int32),
            pltpu.VMEM((SCH,), jnp.int32),
            pltpu.VMEM((SCH,), jnp.float32),
            pltpu.VMEM((N_PAD,), jnp.float32),
            pltpu.VMEM((C, D), jnp.float32),
            pltpu.VMEM((C, D), jnp.float32),
            pltpu.VMEM((112,), jnp.float32),
            pltpu.VMEM((STRIPE,), jnp.float32),
            pltpu.VMEM_SHARED((R_PAD, D), jnp.float32),
            pltpu.VMEM_SHARED((N_PAD,), jnp.float32),
            pltpu.VMEM_SHARED((N_PAD, D), jnp.float32),
        ],
    )(xw_cat, rw_cat, dst2d, srcf, etf)


def kernel(x, edge_index, edge_type, rel_embed, w_loop, w_in, w_out, w_rel,
           loop_rel, bias, bn_gamma, bn_beta):
    nb = 10
    xw_in, xw_out, loop_res = pl.pallas_call(
        _mm_kernel,
        grid=(nb,),
        in_specs=[
            pl.BlockSpec((N_NODES // nb, D), lambda i: (i, 0)),
            pl.BlockSpec((1, D), lambda i: (0, 0)),
            pl.BlockSpec((D, D), lambda i: (0, 0)),
            pl.BlockSpec((D, D), lambda i: (0, 0)),
            pl.BlockSpec((D, D), lambda i: (0, 0)),
        ],
        out_specs=[
            pl.BlockSpec((N_NODES // nb, D), lambda i: (i, 0)),
            pl.BlockSpec((N_NODES // nb, D), lambda i: (i, 0)),
            pl.BlockSpec((N_NODES // nb, D), lambda i: (i, 0)),
        ],
        out_shape=[jax.ShapeDtypeStruct((N_NODES, D), jnp.float32)] * 3,
    )(x, loop_rel, w_loop, w_in, w_out)

    rw_in, rw_out, rel_out = pl.pallas_call(
        _rel_kernel,
        out_shape=[jax.ShapeDtypeStruct((rel_embed.shape[0], D), jnp.float32)] * 3,
    )(rel_embed, w_in, w_out, w_rel)

    half_off = jnp.array([0, N_NODES], jnp.int32)[:, None, None]
    srcf = (edge_index[1].reshape(2, 16, E_TILE)
            + half_off).reshape(2, 16, NSUP, SCH)
    etf = edge_type.reshape(2, 16, NSUP, SCH)
    dst2d = edge_index[0].reshape(2, 16, NSUP, NSUB, C)
    xw_cat = jnp.concatenate([xw_in, xw_out], axis=0)
    nrel = rel_embed.shape[0]
    rw2 = jnp.zeros((2, R_PAD, D), jnp.float32)
    rw2 = rw2.at[0, :nrel].set(rw_in).at[1, :nrel].set(rw_out)

    acc = _sc_call(xw_cat, rw2, dst2d, srcf, etf)

    out = pl.pallas_call(
        _bn_kernel,
        out_shape=jax.ShapeDtypeStruct((N_NODES, D), jnp.float32),
    )(acc, loop_res, bias, bn_gamma, bn_beta)
    return out, rel_out
